# baseline (speedup 1.0000x reference)
import jax
import jax.numpy as jnp
from jax.experimental import pallas as pl
from jax.experimental.pallas import tpu as pltpu

N = 10000
D = 512
TILE_M = 400
TILE_T = 1000


def _xform_kernel(h_ref, w_ref, b_ref, t_ref):
    acc = jnp.dot(h_ref[...], w_ref[...], preferred_element_type=jnp.float32)
    t_ref[...] = (acc + b_ref[...]).astype(jnp.bfloat16)


def _agg_kernel(hg_ref, t_ref, out_ref):
    acc = jnp.dot(hg_ref[...], t_ref[...], preferred_element_type=jnp.float32)
    out_ref[...] = jnp.maximum(acc, 0.0)


def _layer(h, hg16, W, b):
    t = pl.pallas_call(
        _xform_kernel,
        grid=(N // TILE_T,),
        in_specs=[
            pl.BlockSpec((TILE_T, D), lambda i: (i, 0)),
            pl.BlockSpec((D, D), lambda i: (0, 0)),
            pl.BlockSpec((1, D), lambda i: (0, 0)),
        ],
        out_specs=pl.BlockSpec((TILE_T, D), lambda i: (i, 0)),
        out_shape=jax.ShapeDtypeStruct((N, D), jnp.bfloat16),
        compiler_params=pltpu.CompilerParams(
            dimension_semantics=("parallel",)),
    )(h, W, b.reshape(1, D))
    return pl.pallas_call(
        _agg_kernel,
        grid=(N // TILE_M,),
        in_specs=[
            pl.BlockSpec((TILE_M, N), lambda i: (i, 0)),
            pl.BlockSpec((N, D), lambda i: (0, 0)),
        ],
        out_specs=pl.BlockSpec((TILE_M, D), lambda i: (i, 0)),
        out_shape=jax.ShapeDtypeStruct((N, D), jnp.float32),
        compiler_params=pltpu.CompilerParams(
            dimension_semantics=("parallel",)),
    )(hg16, t)


def kernel(x, hg, W1, b1, W2, b2, W3, b3):
    hg16 = hg.astype(jnp.bfloat16)
    h = _layer(x, hg16, W1, b1)
    h = _layer(h, hg16, W2, b2)
    h = _layer(h, hg16, W3, b3)
    return h


# speedup vs baseline: 1.1707x; 1.1707x over previous
import jax
import jax.numpy as jnp
from jax.experimental import pallas as pl
from jax.experimental.pallas import tpu as pltpu

N = 10000
D = 512
TILE_M1 = 200
TILE_M = 400
TILE_T = 1000


def _xform_kernel(h_ref, w_ref, b_ref, t_ref):
    acc = jnp.dot(h_ref[...], w_ref[...], preferred_element_type=jnp.float32)
    t_ref[...] = (acc + b_ref[...]).astype(jnp.bfloat16)


def _agg_cast_kernel(hg_ref, t_ref, out_ref, hg16_ref):
    hg16 = hg_ref[...].astype(jnp.bfloat16)
    hg16_ref[...] = hg16
    acc = jnp.dot(hg16, t_ref[...], preferred_element_type=jnp.float32)
    out_ref[...] = jnp.maximum(acc, 0.0)


def _agg_kernel(hg16_ref, t_ref, out_ref):
    acc = jnp.dot(hg16_ref[...], t_ref[...],
                  preferred_element_type=jnp.float32)
    out_ref[...] = jnp.maximum(acc, 0.0)


def _xform(h, W, b):
    return pl.pallas_call(
        _xform_kernel,
        grid=(N // TILE_T,),
        in_specs=[
            pl.BlockSpec((TILE_T, D), lambda i: (i, 0)),
            pl.BlockSpec((D, D), lambda i: (0, 0)),
            pl.BlockSpec((1, D), lambda i: (0, 0)),
        ],
        out_specs=pl.BlockSpec((TILE_T, D), lambda i: (i, 0)),
        out_shape=jax.ShapeDtypeStruct((N, D), jnp.bfloat16),
        compiler_params=pltpu.CompilerParams(
            dimension_semantics=("parallel",)),
    )(h, W, b.reshape(1, D))


def kernel(x, hg, W1, b1, W2, b2, W3, b3):
    t1 = _xform(x, W1, b1)
    h1, hg16 = pl.pallas_call(
        _agg_cast_kernel,
        grid=(N // TILE_M1,),
        in_specs=[
            pl.BlockSpec((TILE_M1, N), lambda i: (i, 0)),
            pl.BlockSpec((N, D), lambda i: (0, 0)),
        ],
        out_specs=[
            pl.BlockSpec((TILE_M1, D), lambda i: (i, 0)),
            pl.BlockSpec((TILE_M1, N), lambda i: (i, 0)),
        ],
        out_shape=[
            jax.ShapeDtypeStruct((N, D), jnp.float32),
            jax.ShapeDtypeStruct((N, N), jnp.bfloat16),
        ],
        compiler_params=pltpu.CompilerParams(
            dimension_semantics=("parallel",)),
    )(hg, t1)

    h = h1
    for W, b in ((W2, b2), (W3, b3)):
        t = _xform(h, W, b)
        h = pl.pallas_call(
            _agg_kernel,
            grid=(N // TILE_M,),
            in_specs=[
                pl.BlockSpec((TILE_M, N), lambda i: (i, 0)),
                pl.BlockSpec((N, D), lambda i: (0, 0)),
            ],
            out_specs=pl.BlockSpec((TILE_M, D), lambda i: (i, 0)),
            out_shape=jax.ShapeDtypeStruct((N, D), jnp.float32),
            compiler_params=pltpu.CompilerParams(
                dimension_semantics=("parallel",)),
        )(hg16, t)
    return h
